# baseline (speedup 1.0000x reference)
import functools

import jax
import jax.numpy as jnp
from jax.experimental import pallas as pl
from jax.experimental.pallas import tpu as pltpu

E = 16
K = 2
T = 128
D = 1024
F = 2048

EB = 1
NE = E // EB


def _silu(g):
    return g * jax.nn.sigmoid(g)


def _moe_kernel(route_ref, rw_ref, x_ref, w1_ref, w3_ref, w2_ref, out_ref):
    i = pl.program_id(0)

    @pl.when(i == 0)
    def _():
        out_ref[:, :] = jnp.zeros_like(out_ref)

    x = x_ref[:, :].astype(jnp.bfloat16)

    def mm(a, b):
        return jax.lax.dot_general(
            a, b.astype(jnp.bfloat16),
            (((1,), (1,)), ((), ())),
            preferred_element_type=jnp.float32)

    for s in range(EB):
        e = i * EB + s
        y = jnp.zeros((T, D), jnp.float32)
        for jb in range(4):
            fs = jb * (F // 4)
            g = mm(x, w1_ref[s, 0, pl.ds(fs, F // 4), :])
            u = mm(x, w3_ref[s, 0, pl.ds(fs, F // 4), :])
            h = (_silu(g) * u).astype(jnp.bfloat16)
            y += mm(h, w2_ref[s, :, pl.ds(fs, F // 4)])

        sel = (route_ref[:, :] == e).astype(jnp.float32)
        combine = jnp.sum(sel * rw_ref[:, :], axis=1, keepdims=True)

        out_ref[:, :] += combine * y


@jax.jit
def kernel(hidden_states, expert_routing_table, router_weights, w13, w2):
    route = expert_routing_table.astype(jnp.int32)
    w13r = w13.reshape(E, 2, F, D)

    out = pl.pallas_call(
        _moe_kernel,
        grid=(NE,),
        in_specs=[
            pl.BlockSpec((T, K), lambda i: (0, 0)),
            pl.BlockSpec((T, K), lambda i: (0, 0)),
            pl.BlockSpec((T, D), lambda i: (0, 0)),
            pl.BlockSpec((EB, 1, F, D), lambda i: (i, 0, 0, 0)),
            pl.BlockSpec((EB, 1, F, D), lambda i: (i, 1, 0, 0)),
            pl.BlockSpec((EB, D, F), lambda i: (i, 0, 0)),
        ],
        out_specs=pl.BlockSpec((T, D), lambda i: (0, 0)),
        out_shape=jax.ShapeDtypeStruct((T, D), jnp.float32),
        compiler_params=pltpu.CompilerParams(
            dimension_semantics=("arbitrary",),
            vmem_limit_bytes=60 * 1024 * 1024,
        ),
    )(route, router_weights, hidden_states, w13r, w13r, w2)
    return out
